# baseline (speedup 1.0000x reference)
import functools

import jax
import jax.numpy as jnp
from jax import lax
from jax.experimental import pallas as pl
from jax.experimental.pallas import tpu as pltpu
from jax.experimental.pallas import tpu_sc as plsc

B, L = 1024, 200
H = 128
NHALF = 2
BH = B // NHALF
NT = BH * L
NC, NS = 2, 16
NW = NC * NS
TOK_PER_W = NT // NW
CHUNK = 128
NCHUNK = TOK_PER_W // CHUNK
NBUF = 5
AHEAD = 2
EPS = 1e-12
BB = 32
NSTEP = BH // BB


def _sc_gather_body(ids_hbm, wtab_hbm, out_hbm, idxall, *bufs_and_sems):
    bufs = bufs_and_sems[:NBUF]
    gsems = bufs_and_sems[NBUF:2 * NBUF]
    ssems = bufs_and_sems[2 * NBUF:]
    wid = lax.axis_index("s") * NC + lax.axis_index("c")
    base = wid * TOK_PER_W

    pltpu.sync_copy(ids_hbm.at[wid], idxall)

    for c in range(AHEAD):
        pltpu.async_copy(wtab_hbm.at[idxall.at[c]], bufs[c], gsems[c])

    @pl.loop(0, NCHUNK // NBUF)
    def _superiter(gi):
        for b in range(NBUF):
            c = gi * NBUF + b
            pltpu.make_async_copy(
                wtab_hbm.at[pl.ds(0, CHUNK)], bufs[b], gsems[b]).wait()

            b2 = (b + AHEAD) % NBUF

            @pl.when(c + AHEAD < NCHUNK)
            def _():
                @pl.when(c + AHEAD >= NBUF)
                def _():
                    pltpu.make_async_copy(
                        bufs[b2], out_hbm.at[pl.ds(base, CHUNK)],
                        ssems[b2]).wait()
                pltpu.async_copy(
                    wtab_hbm.at[idxall.at[c + AHEAD]], bufs[b2], gsems[b2])

            pltpu.async_copy(
                bufs[b], out_hbm.at[pl.ds(base + c * CHUNK, CHUNK)],
                ssems[b])

    for c in range(NCHUNK - NBUF, NCHUNK):
        b = c % NBUF
        pltpu.make_async_copy(
            bufs[b], out_hbm.at[pl.ds(base + c * CHUNK, CHUNK)],
            ssems[b]).wait()


_sc_gather = pl.kernel(
    _sc_gather_body,
    out_type=jax.ShapeDtypeStruct((NT, H), jnp.float32),
    mesh=plsc.VectorSubcoreMesh(
        core_axis_name="c", subcore_axis_name="s",
        num_cores=NC, num_subcores=NS),
    scratch_types=(
        [pltpu.VMEM((NCHUNK, CHUNK), jnp.int32)]
        + [pltpu.VMEM((CHUNK, H), jnp.float32) for _ in range(NBUF)]
        + [pltpu.SemaphoreType.DMA for _ in range(2 * NBUF)]
    ),
)


def _tc_ln_body(acc_ref, x_ref, pos_ref, g_ref, b_ref, o_ref):
    del acc_ref
    x = x_ref[...] + pos_ref[...][None, :, :]
    mean = jnp.mean(x, axis=-1, keepdims=True)
    xc = x - mean
    var = jnp.mean(xc * xc, axis=-1, keepdims=True)
    normed = xc * lax.rsqrt(var + EPS)
    o_ref[...] = normed * g_ref[0][None, None, :] + b_ref[0][None, None, :]


def _make_tc_ln(half):
    off = half * NSTEP
    return pl.pallas_call(
        _tc_ln_body,
        grid=(NSTEP,),
        in_specs=[
            pl.BlockSpec(memory_space=pl.ANY),
            pl.BlockSpec((BB, L, H), lambda i: (i, 0, 0)),
            pl.BlockSpec((L, H), lambda i: (0, 0)),
            pl.BlockSpec((1, H), lambda i: (0, 0)),
            pl.BlockSpec((1, H), lambda i: (0, 0)),
        ],
        out_specs=pl.BlockSpec((BB, L, H), lambda i: (i + off, 0, 0)),
        out_shape=jax.ShapeDtypeStruct((B, L, H), jnp.float32),
        input_output_aliases={0: 0},
    )


_tc_ln_halves = tuple(_make_tc_ln(h) for h in range(NHALF))


@jax.jit
def kernel(input_ids, word_table, pos_table, gamma, beta):
    ids = input_ids.astype(jnp.int32).reshape(NHALF, NW, NCHUNK, CHUNK)
    pos = pos_table[:L]
    g2 = gamma.reshape(1, H)
    b2 = beta.reshape(1, H)

    gathered = [_sc_gather(ids[h], word_table) for h in range(NHALF)]

    out = jnp.zeros((B, L, H), jnp.float32)
    for h in range(NHALF):
        out = _tc_ln_halves[h](
            out, gathered[h].reshape(BH, L, H), pos, g2, b2)
    return out


# speedup vs baseline: 1.2513x; 1.2513x over previous
import functools

import jax
import jax.numpy as jnp
from jax import lax
from jax.experimental import pallas as pl
from jax.experimental.pallas import tpu as pltpu
from jax.experimental.pallas import tpu_sc as plsc

B, L = 1024, 200
H = 128
NHALF = 2
BH = B // NHALF
NT = BH * L
NC, NS = 2, 16
NW = NC * NS
TOK_PER_W = NT // NW
CHUNK = 128
NCHUNK = TOK_PER_W // CHUNK
NBUF = 5
AHEAD = 2
EPS = 1e-12
BB = 32
NSTEP = BH // BB


def _sc_gather_body(ids_hbm, wtab_hbm, out_hbm, idxall, *bufs_and_sems):
    bufs = bufs_and_sems[:NBUF]
    gsems = bufs_and_sems[NBUF:2 * NBUF]
    ssems = bufs_and_sems[2 * NBUF:]
    wid = lax.axis_index("s") * NC + lax.axis_index("c")
    base = wid * TOK_PER_W

    pltpu.sync_copy(ids_hbm.at[wid], idxall)

    for c in range(AHEAD):
        pltpu.async_copy(wtab_hbm.at[idxall.at[c]], bufs[c], gsems[c])

    @pl.loop(0, NCHUNK // NBUF)
    def _superiter(gi):
        for b in range(NBUF):
            c = gi * NBUF + b
            pltpu.make_async_copy(
                wtab_hbm.at[pl.ds(0, CHUNK)], bufs[b], gsems[b]).wait()

            b2 = (b + AHEAD) % NBUF

            @pl.when(c + AHEAD < NCHUNK)
            def _():
                @pl.when(c + AHEAD >= NBUF)
                def _():
                    pltpu.make_async_copy(
                        bufs[b2], out_hbm.at[pl.ds(base, CHUNK)],
                        ssems[b2]).wait()
                pltpu.async_copy(
                    wtab_hbm.at[idxall.at[c + AHEAD]], bufs[b2], gsems[b2])

            pltpu.async_copy(
                bufs[b], out_hbm.at[pl.ds(base + c * CHUNK, CHUNK)],
                ssems[b])

    for c in range(NCHUNK - NBUF, NCHUNK):
        b = c % NBUF
        pltpu.make_async_copy(
            bufs[b], out_hbm.at[pl.ds(base + c * CHUNK, CHUNK)],
            ssems[b]).wait()


_sc_gather = pl.kernel(
    _sc_gather_body,
    out_type=jax.ShapeDtypeStruct((NT, H), jnp.float32),
    mesh=plsc.VectorSubcoreMesh(
        core_axis_name="c", subcore_axis_name="s",
        num_cores=NC, num_subcores=NS),
    scratch_types=(
        [pltpu.VMEM((NCHUNK, CHUNK), jnp.int32)]
        + [pltpu.VMEM((CHUNK, H), jnp.float32) for _ in range(NBUF)]
        + [pltpu.SemaphoreType.DMA for _ in range(2 * NBUF)]
    ),
)


def _tc_ln_body(acc_ref, x_ref, pos_ref, g_ref, b_ref, o_ref):
    del acc_ref
    x = x_ref[...] + pos_ref[...][None, :, :]
    mean = jnp.mean(x, axis=-1, keepdims=True)
    xc = x - mean
    var = jnp.mean(xc * xc, axis=-1, keepdims=True)
    normed = xc * lax.rsqrt(var + EPS)
    o_ref[...] = normed * g_ref[0][None, None, :] + b_ref[0][None, None, :]


def _tc_ln_first_body(x_ref, pos_ref, g_ref, b_ref, o_ref):
    _tc_ln_body(None, x_ref, pos_ref, g_ref, b_ref, o_ref)


_tc_ln_first = pl.pallas_call(
    _tc_ln_first_body,
    grid=(NSTEP,),
    in_specs=[
        pl.BlockSpec((BB, L, H), lambda i: (i, 0, 0)),
        pl.BlockSpec((L, H), lambda i: (0, 0)),
        pl.BlockSpec((1, H), lambda i: (0, 0)),
        pl.BlockSpec((1, H), lambda i: (0, 0)),
    ],
    out_specs=pl.BlockSpec((BB, L, H), lambda i: (i, 0, 0)),
    out_shape=jax.ShapeDtypeStruct((B, L, H), jnp.float32),
)

_tc_ln_second = pl.pallas_call(
    _tc_ln_body,
    grid=(NSTEP,),
    in_specs=[
        pl.BlockSpec(memory_space=pl.ANY),
        pl.BlockSpec((BB, L, H), lambda i: (i, 0, 0)),
        pl.BlockSpec((L, H), lambda i: (0, 0)),
        pl.BlockSpec((1, H), lambda i: (0, 0)),
        pl.BlockSpec((1, H), lambda i: (0, 0)),
    ],
    out_specs=pl.BlockSpec((BB, L, H), lambda i: (i + NSTEP, 0, 0)),
    out_shape=jax.ShapeDtypeStruct((B, L, H), jnp.float32),
    input_output_aliases={0: 0},
)


@jax.jit
def kernel(input_ids, word_table, pos_table, gamma, beta):
    ids = input_ids.astype(jnp.int32).reshape(NHALF, NW, NCHUNK, CHUNK)
    pos = pos_table[:L]
    g2 = gamma.reshape(1, H)
    b2 = beta.reshape(1, H)

    gathered = [_sc_gather(ids[h], word_table) for h in range(NHALF)]

    out = _tc_ln_first(gathered[0].reshape(BH, L, H), pos, g2, b2)
    return _tc_ln_second(out, gathered[1].reshape(BH, L, H), pos, g2, b2)


# speedup vs baseline: 1.2569x; 1.0045x over previous
import functools

import jax
import jax.numpy as jnp
from jax import lax
from jax.experimental import pallas as pl
from jax.experimental.pallas import tpu as pltpu
from jax.experimental.pallas import tpu_sc as plsc

B, L = 1024, 200
H = 128
NHALF = 2
BH = B // NHALF
NT = BH * L
NC, NS = 2, 16
NW = NC * NS
TOK_PER_W = NT // NW
CHUNK = 128
NCHUNK = TOK_PER_W // CHUNK
NBUF = 5
AHEAD = 3
EPS = 1e-12
BB = 32
NSTEP = BH // BB


def _sc_gather_body(ids_hbm, wtab_hbm, out_hbm, idxall, *bufs_and_sems):
    bufs = bufs_and_sems[:NBUF]
    gsems = bufs_and_sems[NBUF:2 * NBUF]
    ssems = bufs_and_sems[2 * NBUF:]
    wid = lax.axis_index("s") * NC + lax.axis_index("c")
    base = wid * TOK_PER_W

    pltpu.sync_copy(ids_hbm.at[wid], idxall)

    for c in range(AHEAD):
        pltpu.async_copy(wtab_hbm.at[idxall.at[c]], bufs[c], gsems[c])

    @pl.loop(0, NCHUNK // NBUF)
    def _superiter(gi):
        for b in range(NBUF):
            c = gi * NBUF + b
            pltpu.make_async_copy(
                wtab_hbm.at[pl.ds(0, CHUNK)], bufs[b], gsems[b]).wait()

            b2 = (b + AHEAD) % NBUF

            @pl.when(c + AHEAD < NCHUNK)
            def _():
                @pl.when(c + AHEAD >= NBUF)
                def _():
                    pltpu.make_async_copy(
                        bufs[b2], out_hbm.at[pl.ds(base, CHUNK)],
                        ssems[b2]).wait()
                pltpu.async_copy(
                    wtab_hbm.at[idxall.at[c + AHEAD]], bufs[b2], gsems[b2])

            pltpu.async_copy(
                bufs[b], out_hbm.at[pl.ds(base + c * CHUNK, CHUNK)],
                ssems[b])

    for c in range(NCHUNK - NBUF, NCHUNK):
        b = c % NBUF
        pltpu.make_async_copy(
            bufs[b], out_hbm.at[pl.ds(base + c * CHUNK, CHUNK)],
            ssems[b]).wait()


_sc_gather = pl.kernel(
    _sc_gather_body,
    out_type=jax.ShapeDtypeStruct((NT, H), jnp.float32),
    mesh=plsc.VectorSubcoreMesh(
        core_axis_name="c", subcore_axis_name="s",
        num_cores=NC, num_subcores=NS),
    scratch_types=(
        [pltpu.VMEM((NCHUNK, CHUNK), jnp.int32)]
        + [pltpu.VMEM((CHUNK, H), jnp.float32) for _ in range(NBUF)]
        + [pltpu.SemaphoreType.DMA for _ in range(2 * NBUF)]
    ),
)


def _tc_ln_body(acc_ref, x_ref, pos_ref, g_ref, b_ref, o_ref):
    del acc_ref
    x = x_ref[...] + pos_ref[...][None, :, :]
    mean = jnp.mean(x, axis=-1, keepdims=True)
    xc = x - mean
    var = jnp.mean(xc * xc, axis=-1, keepdims=True)
    normed = xc * lax.rsqrt(var + EPS)
    o_ref[...] = normed * g_ref[0][None, None, :] + b_ref[0][None, None, :]


def _tc_ln_first_body(x_ref, pos_ref, g_ref, b_ref, o_ref):
    _tc_ln_body(None, x_ref, pos_ref, g_ref, b_ref, o_ref)


_tc_ln_first = pl.pallas_call(
    _tc_ln_first_body,
    grid=(NSTEP,),
    in_specs=[
        pl.BlockSpec((BB, L, H), lambda i: (i, 0, 0)),
        pl.BlockSpec((L, H), lambda i: (0, 0)),
        pl.BlockSpec((1, H), lambda i: (0, 0)),
        pl.BlockSpec((1, H), lambda i: (0, 0)),
    ],
    out_specs=pl.BlockSpec((BB, L, H), lambda i: (i, 0, 0)),
    out_shape=jax.ShapeDtypeStruct((B, L, H), jnp.float32),
)

_tc_ln_second = pl.pallas_call(
    _tc_ln_body,
    grid=(NSTEP,),
    in_specs=[
        pl.BlockSpec(memory_space=pl.ANY),
        pl.BlockSpec((BB, L, H), lambda i: (i, 0, 0)),
        pl.BlockSpec((L, H), lambda i: (0, 0)),
        pl.BlockSpec((1, H), lambda i: (0, 0)),
        pl.BlockSpec((1, H), lambda i: (0, 0)),
    ],
    out_specs=pl.BlockSpec((BB, L, H), lambda i: (i + NSTEP, 0, 0)),
    out_shape=jax.ShapeDtypeStruct((B, L, H), jnp.float32),
    input_output_aliases={0: 0},
)


@jax.jit
def kernel(input_ids, word_table, pos_table, gamma, beta):
    ids = input_ids.astype(jnp.int32).reshape(NHALF, NW, NCHUNK, CHUNK)
    pos = pos_table[:L]
    g2 = gamma.reshape(1, H)
    b2 = beta.reshape(1, H)

    gathered = [_sc_gather(ids[h], word_table) for h in range(NHALF)]

    out = _tc_ln_first(gathered[0].reshape(BH, L, H), pos, g2, b2)
    return _tc_ln_second(out, gathered[1].reshape(BH, L, H), pos, g2, b2)


# speedup vs baseline: 1.2635x; 1.0052x over previous
import functools

import jax
import jax.numpy as jnp
from jax import lax
from jax.experimental import pallas as pl
from jax.experimental.pallas import tpu as pltpu
from jax.experimental.pallas import tpu_sc as plsc

B, L = 1024, 200
H = 128
NHALF = 2
BH = B // NHALF
NT = BH * L
NC, NS = 2, 16
NW = NC * NS
TOK_PER_W = NT // NW
CHUNK = 128
NCHUNK = TOK_PER_W // CHUNK
NBUF = 5
AHEAD = 4
EPS = 1e-12
BB = 32
NSTEP = BH // BB


def _sc_gather_body(ids_hbm, wtab_hbm, out_hbm, idxall, *bufs_and_sems):
    bufs = bufs_and_sems[:NBUF]
    gsems = bufs_and_sems[NBUF:2 * NBUF]
    ssems = bufs_and_sems[2 * NBUF:]
    wid = lax.axis_index("s") * NC + lax.axis_index("c")
    base = wid * TOK_PER_W

    pltpu.sync_copy(ids_hbm.at[wid], idxall)

    for c in range(AHEAD):
        pltpu.async_copy(wtab_hbm.at[idxall.at[c]], bufs[c], gsems[c])

    @pl.loop(0, NCHUNK // NBUF)
    def _superiter(gi):
        for b in range(NBUF):
            c = gi * NBUF + b
            pltpu.make_async_copy(
                wtab_hbm.at[pl.ds(0, CHUNK)], bufs[b], gsems[b]).wait()

            b2 = (b + AHEAD) % NBUF

            @pl.when(c + AHEAD < NCHUNK)
            def _():
                @pl.when(c + AHEAD >= NBUF)
                def _():
                    pltpu.make_async_copy(
                        bufs[b2], out_hbm.at[pl.ds(base, CHUNK)],
                        ssems[b2]).wait()
                pltpu.async_copy(
                    wtab_hbm.at[idxall.at[c + AHEAD]], bufs[b2], gsems[b2])

            pltpu.async_copy(
                bufs[b], out_hbm.at[pl.ds(base + c * CHUNK, CHUNK)],
                ssems[b])

    for c in range(NCHUNK - NBUF, NCHUNK):
        b = c % NBUF
        pltpu.make_async_copy(
            bufs[b], out_hbm.at[pl.ds(base + c * CHUNK, CHUNK)],
            ssems[b]).wait()


_sc_gather = pl.kernel(
    _sc_gather_body,
    out_type=jax.ShapeDtypeStruct((NT, H), jnp.float32),
    mesh=plsc.VectorSubcoreMesh(
        core_axis_name="c", subcore_axis_name="s",
        num_cores=NC, num_subcores=NS),
    scratch_types=(
        [pltpu.VMEM((NCHUNK, CHUNK), jnp.int32)]
        + [pltpu.VMEM((CHUNK, H), jnp.float32) for _ in range(NBUF)]
        + [pltpu.SemaphoreType.DMA for _ in range(2 * NBUF)]
    ),
)


def _tc_ln_body(acc_ref, x_ref, pos_ref, g_ref, b_ref, o_ref):
    del acc_ref
    x = x_ref[...] + pos_ref[...][None, :, :]
    mean = jnp.mean(x, axis=-1, keepdims=True)
    xc = x - mean
    var = jnp.mean(xc * xc, axis=-1, keepdims=True)
    normed = xc * lax.rsqrt(var + EPS)
    o_ref[...] = normed * g_ref[0][None, None, :] + b_ref[0][None, None, :]


def _tc_ln_first_body(x_ref, pos_ref, g_ref, b_ref, o_ref):
    _tc_ln_body(None, x_ref, pos_ref, g_ref, b_ref, o_ref)


_tc_ln_first = pl.pallas_call(
    _tc_ln_first_body,
    grid=(NSTEP,),
    in_specs=[
        pl.BlockSpec((BB, L, H), lambda i: (i, 0, 0)),
        pl.BlockSpec((L, H), lambda i: (0, 0)),
        pl.BlockSpec((1, H), lambda i: (0, 0)),
        pl.BlockSpec((1, H), lambda i: (0, 0)),
    ],
    out_specs=pl.BlockSpec((BB, L, H), lambda i: (i, 0, 0)),
    out_shape=jax.ShapeDtypeStruct((B, L, H), jnp.float32),
)

_tc_ln_second = pl.pallas_call(
    _tc_ln_body,
    grid=(NSTEP,),
    in_specs=[
        pl.BlockSpec(memory_space=pl.ANY),
        pl.BlockSpec((BB, L, H), lambda i: (i, 0, 0)),
        pl.BlockSpec((L, H), lambda i: (0, 0)),
        pl.BlockSpec((1, H), lambda i: (0, 0)),
        pl.BlockSpec((1, H), lambda i: (0, 0)),
    ],
    out_specs=pl.BlockSpec((BB, L, H), lambda i: (i + NSTEP, 0, 0)),
    out_shape=jax.ShapeDtypeStruct((B, L, H), jnp.float32),
    input_output_aliases={0: 0},
)


@jax.jit
def kernel(input_ids, word_table, pos_table, gamma, beta):
    ids = input_ids.astype(jnp.int32).reshape(NHALF, NW, NCHUNK, CHUNK)
    pos = pos_table[:L]
    g2 = gamma.reshape(1, H)
    b2 = beta.reshape(1, H)

    gathered = [_sc_gather(ids[h], word_table) for h in range(NHALF)]

    out = _tc_ln_first(gathered[0].reshape(BH, L, H), pos, g2, b2)
    return _tc_ln_second(out, gathered[1].reshape(BH, L, H), pos, g2, b2)
